# Initial kernel scaffold; baseline (speedup 1.0000x reference)
import jax
import jax.numpy as jnp
from jax.experimental import pallas as pl


def kernel(x, edge_index, W1l, b1, W1r, W2l, b2, W2r):
    raise NotImplementedError("write your pallas kernel here")


# speedup vs baseline: 12.8566x; 12.8566x over previous
import functools

import jax
import jax.numpy as jnp
from jax import lax
from jax.experimental import pallas as pl
from jax.experimental.pallas import tpu as pltpu
from jax.experimental.pallas import tpu_sc as plsc

N_NODES = 10000
D_IN = 128
D_HID = 16
D_OUT = 128

NC = 2
NS = 16
LANES = 16
CHUNK = 128
N_ACC = 10240
ROWS_PER_SUB = N_ACC // NS

_mesh = plsc.VectorSubcoreMesh(core_axis_name="c", subcore_axis_name="s")
_sc_params = pltpu.CompilerParams(use_tc_tiling_on_sc=False)


def _make_seg_sum(n_chunks, with_counts):
    out_types = [jax.ShapeDtypeStruct((NC, N_ACC, LANES), jnp.float32)]
    scratch = [
        pltpu.VMEM((n_chunks, CHUNK), jnp.int32),
        pltpu.VMEM((n_chunks, CHUNK), jnp.int32),
        pltpu.VMEM((CHUNK, LANES), jnp.float32),
        pltpu.VMEM((CHUNK, LANES), jnp.float32),
        pltpu.VMEM_SHARED((N_ACC, LANES), jnp.float32),
        pltpu.VMEM_SHARED((N_ACC, LANES), jnp.float32),
        pltpu.SemaphoreType.DMA,
    ]
    if with_counts:
        out_types.append(jax.ShapeDtypeStruct((NC, N_ACC, LANES), jnp.float32))

        @functools.partial(pl.kernel, out_type=out_types, mesh=_mesh,
                           scratch_types=scratch,
                           compiler_params=_sc_params)
        def seg(feat_hbm, src_hbm, dst_hbm, zero_hbm, ones_hbm,
                out_hbm, cnt_hbm,
                src_v, dst_v, rows_v, ones_v, acc_sh, cnt_sh, sem):
            c = lax.axis_index("c")
            s = lax.axis_index("s")
            r0 = s * ROWS_PER_SUB
            rows = pl.ds(r0, ROWS_PER_SUB)
            pltpu.sync_copy(zero_hbm.at[rows], acc_sh.at[rows])
            pltpu.sync_copy(zero_hbm.at[rows], cnt_sh.at[rows])
            pltpu.sync_copy(src_hbm.at[c].at[s], src_v)
            pltpu.sync_copy(dst_hbm.at[c].at[s], dst_v)
            pltpu.sync_copy(ones_hbm, ones_v)
            plsc.subcore_barrier()

            @pl.loop(0, n_chunks)
            def _(j):
                pltpu.async_copy(feat_hbm.at[src_v.at[j]], rows_v, sem).wait()
                pltpu.sync_copy(rows_v, acc_sh.at[dst_v.at[j]], add=True)
                pltpu.sync_copy(ones_v, cnt_sh.at[dst_v.at[j]], add=True)

            plsc.subcore_barrier()
            pltpu.sync_copy(acc_sh.at[rows], out_hbm.at[c].at[rows])
            pltpu.sync_copy(cnt_sh.at[rows], cnt_hbm.at[c].at[rows])

        return seg

    @functools.partial(pl.kernel, out_type=out_types, mesh=_mesh,
                       scratch_types=scratch,
                           compiler_params=_sc_params)
    def seg(feat_hbm, src_hbm, dst_hbm, zero_hbm, ones_hbm,
            out_hbm,
            src_v, dst_v, rows_v, ones_v, acc_sh, cnt_sh, sem):
        c = lax.axis_index("c")
        s = lax.axis_index("s")
        r0 = s * ROWS_PER_SUB
        rows = pl.ds(r0, ROWS_PER_SUB)
        pltpu.sync_copy(zero_hbm.at[rows], acc_sh.at[rows])
        pltpu.sync_copy(src_hbm.at[c].at[s], src_v)
        pltpu.sync_copy(dst_hbm.at[c].at[s], dst_v)
        plsc.subcore_barrier()

        @pl.loop(0, n_chunks)
        def _(j):
            pltpu.async_copy(feat_hbm.at[src_v.at[j]], rows_v, sem).wait()
            pltpu.sync_copy(rows_v, acc_sh.at[dst_v.at[j]], add=True)

        plsc.subcore_barrier()
        pltpu.sync_copy(acc_sh.at[rows], out_hbm.at[c].at[rows])

    return seg


def _mm_in_body(x_ref, wl_ref, wr_ref, y1_ref, xr_ref):
    x = x_ref[...]
    y1_ref[...] = jnp.dot(x, wl_ref[...], preferred_element_type=jnp.float32)
    xr_ref[...] = jnp.dot(x, wr_ref[...], preferred_element_type=jnp.float32)


def _h_body(p_ref, cp_ref, xr_ref, b1_ref, h_ref, cnt_ref):
    cnt = cp_ref[0] + cp_ref[1]
    mean = (p_ref[0] + p_ref[1]) / jnp.maximum(cnt, 1.0)
    h_ref[...] = jnp.maximum(mean + b1_ref[...] + xr_ref[...], 0.0)
    cnt_ref[...] = cnt


def _out_body(p2_ref, cnt_ref, h_ref, w2l_ref, b2_ref, w2r_ref, o_ref):
    mean2 = (p2_ref[0] + p2_ref[1]) / jnp.maximum(cnt_ref[...], 1.0)
    o_ref[...] = (
        jnp.dot(mean2, w2l_ref[...], preferred_element_type=jnp.float32)
        + b2_ref[...]
        + jnp.dot(h_ref[...], w2r_ref[...], preferred_element_type=jnp.float32)
    )


@jax.jit
def _impl(x, edge_index, W1l, b1, W1r, W2l, b2, W2r):
    src = edge_index[0].astype(jnp.int32)
    dst = edge_index[1].astype(jnp.int32)
    e = src.shape[0]
    n_chunks = -(-e // (NC * NS * CHUNK))
    e_pad = NC * NS * n_chunks * CHUNK
    src3 = jnp.concatenate(
        [src, jnp.zeros((e_pad - e,), jnp.int32)]).reshape(NC, NS, n_chunks, CHUNK)
    dst3 = jnp.concatenate(
        [dst, jnp.full((e_pad - e,), N_NODES, jnp.int32)]).reshape(NC, NS, n_chunks, CHUNK)
    xp = jnp.zeros((N_ACC, D_IN), jnp.float32).at[:N_NODES].set(x)
    zero = jnp.zeros((N_ACC, LANES), jnp.float32)
    ones = jnp.ones((CHUNK, LANES), jnp.float32)

    y1, xr = pl.pallas_call(
        _mm_in_body,
        out_shape=[jax.ShapeDtypeStruct((N_ACC, D_HID), jnp.float32)] * 2,
    )(xp, W1l, W1r)

    seg_c = _make_seg_sum(n_chunks, with_counts=True)
    p1, c1 = seg_c(y1, src3, dst3, zero, ones)

    h, cnt = pl.pallas_call(
        _h_body,
        out_shape=[jax.ShapeDtypeStruct((N_ACC, D_HID), jnp.float32)] * 2,
    )(p1, c1, xr, b1.reshape(1, D_HID))

    seg_p = _make_seg_sum(n_chunks, with_counts=False)
    (p2,) = seg_p(h, src3, dst3, zero, ones)

    out = pl.pallas_call(
        _out_body,
        out_shape=jax.ShapeDtypeStruct((N_ACC, D_OUT), jnp.float32),
    )(p2, cnt, h, W2l, b2.reshape(1, D_OUT), W2r)
    return out[:N_NODES]


def kernel(x, edge_index, W1l, b1, W1r, W2l, b2, W2r):
    return _impl(x, edge_index, W1l, b1, W1r, W2l, b2, W2r)
